# Initial kernel scaffold; baseline (speedup 1.0000x reference)
import jax
import jax.numpy as jnp
from jax.experimental import pallas as pl


def kernel(mem, ent_counter, query, cell_idx):
    raise NotImplementedError("write your pallas kernel here")


# speedup vs baseline: 1.2360x; 1.2360x over previous
import jax
import jax.numpy as jnp
from jax import lax
from jax.experimental import pallas as pl
from jax.experimental.pallas import tpu as pltpu
from jax.experimental.pallas import tpu_sc as plsc

M = 100000
D = 320
B = 16384
NW = 32
OWN = 3200
CCH = 80
L = 16


def _bcast_lane(v, r):
    idx = jnp.full((L,), r, jnp.int32)
    return v.at[idx].get(mode="promise_in_bounds")


def _sc_body(mem_h, cnt_h, query_h, idx_h, out_h,
             idx_v, pos_v, cnt_v, cbuf, qbuf, obuf,
             sem_q, sem_o, sem_s):
    cid = lax.axis_index("c")
    sid = lax.axis_index("s")
    w = sid * 2 + cid
    lo = w * OWN
    rows = jnp.minimum(OWN, M - lo)
    iota = lax.iota(jnp.int32, L)

    def copy_chunk(j, carry):
        r = lo + j * CCH
        pltpu.sync_copy(mem_h.at[pl.ds(r, CCH)], cbuf)
        pltpu.sync_copy(cbuf, out_h.at[pl.ds(r, CCH)])
        return carry

    lax.fori_loop(0, rows // CCH, copy_chunk, 0)

    pltpu.sync_copy(idx_h, idx_v)
    pltpu.sync_copy(cnt_h.at[pl.ds(lo, OWN)], cnt_v)

    def scan_it(i, n):
        c16 = idx_v[pl.ds(i * L, L)]
        owned = (c16 >= lo) & (c16 < lo + OWN)
        inc = plsc.cumsum(owned.astype(jnp.int32))
        offs = n + inc - owned.astype(jnp.int32)
        plsc.store_scatter(pos_v, [offs], i * L + iota, mask=owned)
        return n + jnp.sum(owned.astype(jnp.int32))

    n_owned = lax.fori_loop(0, B // L, scan_it, 0)

    def batch(b, carry):
        base = b * L
        valid = iota < (n_owned - base)
        pos = jnp.where(valid, pos_v[pl.ds(base, L)], 0)
        cells = jnp.where(valid, plsc.load_gather(idx_v, [pos]), lo)

        vcells = jnp.where(valid, cells, -1)
        hit0 = jnp.zeros((L,), jnp.bool_)
        hit1 = jnp.zeros((L,), jnp.bool_)
        for s in range(L):
            cs = _bcast_lane(vcells, s)
            hit0 = hit0 | (lo + iota == cs)
            hit1 = hit1 | (lo + L + iota == cs)
        big = jnp.int32(1 << 30)
        d = jnp.minimum(jnp.min(jnp.where(hit0, big, lo + iota)),
                        jnp.min(jnp.where(hit1, big, lo + L + iota)))

        def round_cond(rem):
            return jnp.any(rem)

        def round_body(rem):
            rcells = jnp.where(rem, cells, -1)
            conflict = jnp.zeros((L,), jnp.bool_)
            for s in range(1, L):
                sh = jnp.maximum(iota - s, 0)
                shifted = rcells.at[sh].get(mode="promise_in_bounds")
                conflict = conflict | ((shifted == cells) & (iota >= s))
            active = rem & jnp.logical_not(conflict)

            cells_t = jnp.where(active, cells, d)
            pos_t = jnp.where(active, pos, 0)
            kv = plsc.load_gather(cnt_v, [cells_t - lo])
            inv = 1.0 / (kv + 1.0)
            av = jnp.where(active, kv * inv, 1.0)
            bv = jnp.where(active, inv, 0.0)
            plsc.store_scatter(cnt_v, [cells_t - lo], kv + 1.0, mask=active)

            cq = pltpu.async_copy(query_h.at[pos_t], qbuf, sem_q)
            co = pltpu.async_copy(out_h.at[cells_t], obuf, sem_o)
            cq.wait()
            co.wait()

            def row_fn(r, carry2):
                ar = _bcast_lane(av, r)
                br = _bcast_lane(bv, r)
                for t in range(D // L):
                    sl = pl.ds(t * L, L)
                    obuf[r, sl] = obuf[r, sl] * ar + qbuf[r, sl] * br
                return carry2

            lax.fori_loop(0, L, row_fn, 0)
            pltpu.async_copy(obuf, out_h.at[cells_t], sem_s).wait()
            return rem & jnp.logical_not(active)

        lax.while_loop(round_cond, round_body, valid)
        return carry

    nb = (n_owned + L - 1) // L
    lax.fori_loop(0, nb, batch, 0)


@jax.jit
def kernel(mem, ent_counter, query, cell_idx):
    cnt_pad = jnp.pad(ent_counter, (0, NW * OWN - M))
    idx32 = cell_idx.astype(jnp.int32)
    mesh = plsc.VectorSubcoreMesh(core_axis_name="c", subcore_axis_name="s",
                                  num_cores=2, num_subcores=16)
    f = pl.kernel(
        _sc_body,
        out_type=jax.ShapeDtypeStruct((M, D), jnp.float32),
        mesh=mesh,
        compiler_params=pltpu.CompilerParams(needs_layout_passes=False, use_tc_tiling_on_sc=False),
        scratch_types=[
            pltpu.VMEM((B,), jnp.int32),
            pltpu.VMEM((B + L,), jnp.int32),
            pltpu.VMEM((OWN,), jnp.float32),
            pltpu.VMEM((CCH, D), jnp.float32),
            pltpu.VMEM((L, D), jnp.float32),
            pltpu.VMEM((L, D), jnp.float32),
            pltpu.SemaphoreType.DMA,
            pltpu.SemaphoreType.DMA,
            pltpu.SemaphoreType.DMA,
        ],
    )
    return f(mem, cnt_pad, query, idx32)


# speedup vs baseline: 2.5265x; 2.0441x over previous
import jax
import jax.numpy as jnp
from jax import lax
from jax.experimental import pallas as pl
from jax.experimental.pallas import tpu as pltpu
from jax.experimental.pallas import tpu_sc as plsc

M = 100000
D = 320
DP = 384
B = 16384
NW = 32
OWN = 3200
CCH = 80
GP = 8
ICH = 2048
L = 16


def _bcast_lane(v, r):
    idx = jnp.full((L,), r, jnp.int32)
    return v.at[idx].get(mode="promise_in_bounds")


def _lane_scalar(v, r):
    iota = lax.iota(jnp.int32, L)
    return jnp.sum(jnp.where(iota == r, v, 0))


def _sc_body(mem_h, cnt_h, q3_h, idx_h, out_h,
             idx_v, pk_v, chpk_v, cnt_v, cb0, cb1, gbuf, gidx,
             si0, si1, so0, so1, sg):
    cid = lax.axis_index("c")
    sid = lax.axis_index("s")
    w = sid * 2 + cid
    lo = w * OWN
    rows = jnp.minimum(OWN, M - lo)
    nch = rows // CCH
    iota = lax.iota(jnp.int32, L)

    pltpu.async_copy(mem_h.at[pl.ds(lo, CCH)], cb0, si0)

    pltpu.sync_copy(cnt_h.at[pl.ds(lo, OWN)], cnt_v)

    def scan_chunk(o, n):
        pltpu.sync_copy(idx_h.at[pl.ds(o * ICH, ICH)], idx_v)

        def scan_it(i, n2):
            c16 = idx_v[pl.ds(i * L, L)]
            owned = (c16 >= lo) & (c16 < lo + OWN)
            inc = plsc.cumsum(owned.astype(jnp.int32))
            offs = n2 + inc - owned.astype(jnp.int32)
            plsc.store_scatter(pk_v, [offs],
                               c16 * (L * 1024) + o * ICH + i * L + iota,
                               mask=owned)
            return n2 + jnp.sum(owned.astype(jnp.int32))

        return lax.fori_loop(0, ICH // L, scan_it, n)

    n_owned = lax.fori_loop(0, B // ICH, scan_chunk, 0)
    nsg = (n_owned + L - 1) // L

    def process(j, cbuf):
        clo = lo + j * CCH

        def sel(g, m):
            pk = pk_v[pl.ds(g * L, L)]
            valid = iota < (n_owned - g * L)
            cells = lax.shift_right_logical(pk, 14)
            inch = valid & (cells >= clo) & (cells < clo + CCH)
            inc = plsc.cumsum(inch.astype(jnp.int32))
            offs = m + inc - inch.astype(jnp.int32)
            plsc.store_scatter(chpk_v, [offs], pk, mask=inch)
            return m + jnp.sum(inch.astype(jnp.int32))

        m = lax.fori_loop(0, nsg, sel, 0)

        def grp(q, carry):
            chp = chpk_v[pl.ds(q * GP, L)]
            lanev = (iota < (m - q * GP)) & (iota < GP)
            pos = jnp.where(lanev, chp & (L * 1024 - 1), 0)
            cells = lax.shift_right_logical(jnp.where(lanev, chp, clo * (L * 1024)), 14)
            lr = cells - clo
            rr = pos & 7
            plsc.store_scatter(gidx, [iota], lax.shift_right_logical(pos, 3),
                               mask=iota < GP)
            pltpu.async_copy(q3_h.at[gidx], gbuf, sg).wait()

            def pair(r, carry2):
                @pl.when(q * GP + r < m)
                def _():
                    csplat = _bcast_lane(cells, r)
                    kv = plsc.load_gather(cnt_v, [csplat - lo])
                    inv = 1.0 / (kv + 1.0)
                    a = kv * inv
                    plsc.store_scatter(cnt_v, [csplat - lo], kv + 1.0,
                                       mask=iota == 0)
                    lr_s = _lane_scalar(lr, r)
                    rr_s = _lane_scalar(rr, r)
                    for t in range(D // L):
                        sl = pl.ds(t * L, L)
                        cbuf[lr_s, sl] = cbuf[lr_s, sl] * a + gbuf[r, rr_s, sl] * inv
                return carry2

            lax.fori_loop(0, GP, pair, 0)
            return carry

        lax.fori_loop(0, (m + GP - 1) // GP, grp, 0)

    bufs = ((cb0, si0, so0), (cb1, si1, so1))

    def outer(jj, carry):
        for bsel in range(2):
            cbuf, si, so = bufs[bsel]
            ocbuf, osi, oso = bufs[1 - bsel]
            j = jj * 2 + bsel
            r0 = lo + j * CCH
            pltpu.make_async_copy(mem_h.at[pl.ds(r0, CCH)], cbuf, si).wait()

            @pl.when(j + 1 < nch)
            def _():
                @pl.when(j >= 1)
                def _():
                    pltpu.make_async_copy(
                        ocbuf, out_h.at[pl.ds(r0 - CCH, CCH)], oso).wait()
                pltpu.async_copy(mem_h.at[pl.ds(r0 + CCH, CCH)], ocbuf, osi)

            process(j, cbuf)
            pltpu.async_copy(cbuf, out_h.at[pl.ds(r0, CCH)], so)
        return carry

    lax.fori_loop(0, nch // 2, outer, 0)

    rlast = lo + (nch - 1) * CCH
    pltpu.make_async_copy(cb0, out_h.at[pl.ds(rlast - CCH, CCH)], so0).wait()
    pltpu.make_async_copy(cb1, out_h.at[pl.ds(rlast, CCH)], so1).wait()


@jax.jit
def kernel(mem, ent_counter, query, cell_idx):
    cnt_pad = jnp.pad(ent_counter, (0, NW * OWN - M))
    q3 = jnp.pad(query, ((0, 0), (0, DP - D))).reshape(B // 8, 8, DP)
    idx32 = cell_idx.astype(jnp.int32)
    mesh = plsc.VectorSubcoreMesh(core_axis_name="c", subcore_axis_name="s",
                                  num_cores=2, num_subcores=16)
    f = pl.kernel(
        _sc_body,
        out_type=jax.ShapeDtypeStruct((M, D), jnp.float32),
        mesh=mesh,
        compiler_params=pltpu.CompilerParams(needs_layout_passes=False),
        scratch_types=[
            pltpu.VMEM((ICH,), jnp.int32),
            pltpu.VMEM((B + L,), jnp.int32),
            pltpu.VMEM((B + L,), jnp.int32),
            pltpu.VMEM((OWN,), jnp.float32),
            pltpu.VMEM((CCH, D), jnp.float32),
            pltpu.VMEM((CCH, D), jnp.float32),
            pltpu.VMEM((GP, 8, DP), jnp.float32),
            pltpu.VMEM((GP,), jnp.int32),
            pltpu.SemaphoreType.DMA,
            pltpu.SemaphoreType.DMA,
            pltpu.SemaphoreType.DMA,
            pltpu.SemaphoreType.DMA,
            pltpu.SemaphoreType.DMA,
        ],
    )
    return f(mem, cnt_pad, q3, idx32)


# speedup vs baseline: 3.1529x; 1.2479x over previous
import jax
import jax.numpy as jnp
from jax import lax
from jax.experimental import pallas as pl
from jax.experimental.pallas import tpu as pltpu
from jax.experimental.pallas import tpu_sc as plsc

M = 100000
D = 320
DP = 384
B = 16384
NW = 32
OWN = 3200
CCH = 80
GP = 4
ICH = 2048
L = 16


def _bcast_lane(v, r):
    idx = jnp.full((L,), r, jnp.int32)
    return v.at[idx].get(mode="promise_in_bounds")


def _lane_scalar(v, r):
    iota = lax.iota(jnp.int32, L)
    return jnp.sum(jnp.where(iota == r, v, 0))


def _sc_body(mem_h, cnt_h, q3_h, idx_h, out_h,
             idx_v, pk_v, chpk_v, cnt_v, cb0, cb1, gb0, gb1, gi0, gi1,
             si0, si1, so0, so1, sg0, sg1):
    cid = lax.axis_index("c")
    sid = lax.axis_index("s")
    w = sid * 2 + cid
    lo = w * OWN
    rows = jnp.minimum(OWN, M - lo)
    nch = rows // CCH
    iota = lax.iota(jnp.int32, L)

    pltpu.async_copy(mem_h.at[pl.ds(lo, CCH)], cb0, si0)

    pltpu.sync_copy(cnt_h.at[pl.ds(lo, OWN)], cnt_v)

    def scan_chunk(o, n):
        pltpu.sync_copy(idx_h.at[pl.ds(o * ICH, ICH)], idx_v)

        def scan_it(i, n2):
            c16 = idx_v[pl.ds(i * L, L)]
            owned = (c16 >= lo) & (c16 < lo + OWN)
            inc = plsc.cumsum(owned.astype(jnp.int32))
            offs = n2 + inc - owned.astype(jnp.int32)
            plsc.store_scatter(pk_v, [offs],
                               c16 * (L * 1024) + o * ICH + i * L + iota,
                               mask=owned)
            return n2 + jnp.sum(owned.astype(jnp.int32))

        return lax.fori_loop(0, ICH // L, scan_it, n)

    n_owned = lax.fori_loop(0, B // ICH, scan_chunk, 0)
    nsg = (n_owned + L - 1) // L

    def process(j, cbuf):
        clo = lo + j * CCH

        def sel(g, m):
            pk = pk_v[pl.ds(g * L, L)]
            valid = iota < (n_owned - g * L)
            cells = lax.shift_right_logical(pk, 14)
            inch = valid & (cells >= clo) & (cells < clo + CCH)
            inc = plsc.cumsum(inch.astype(jnp.int32))
            offs = m + inc - inch.astype(jnp.int32)
            plsc.store_scatter(chpk_v, [offs], pk, mask=inch)
            return m + jnp.sum(inch.astype(jnp.int32))

        m = lax.fori_loop(0, nsg, sel, 0)

        ng = (m + GP - 1) // GP

        def gissue(q, gi, gb, sem):
            chp = chpk_v[pl.ds(q * GP, L)]
            lanev = (iota < (m - q * GP)) & (iota < GP)
            pos = jnp.where(lanev, chp & (L * 1024 - 1), 0)
            plsc.store_scatter(gi, [iota], lax.shift_right_logical(pos, 3),
                               mask=iota < GP)
            pltpu.async_copy(q3_h.at[gi], gb, sem)

        @pl.when(ng > 0)
        def _():
            gissue(0, gi0, gb0, sg0)

        gsets = ((gi0, gb0, sg0), (gi1, gb1, sg1))

        def gouter(qq, carry):
            for gsel in range(2):
                gi, gb, sem = gsets[gsel]
                ogi, ogb, osem = gsets[1 - gsel]
                q = qq * 2 + gsel

                @pl.when(q < ng)
                def _():
                    chp = chpk_v[pl.ds(q * GP, L)]
                    lanev = (iota < (m - q * GP)) & (iota < GP)
                    pos = jnp.where(lanev, chp & (L * 1024 - 1), 0)
                    cells = lax.shift_right_logical(
                        jnp.where(lanev, chp, clo * (L * 1024)), 14)
                    lr = cells - clo
                    rr = pos & 7
                    pltpu.make_async_copy(q3_h.at[gi], gb, sem).wait()

                    @pl.when(q + 1 < ng)
                    def _():
                        gissue(q + 1, ogi, ogb, osem)

                    lrr = lr * 8 + rr

                    def pair(r, carry2):
                        @pl.when(q * GP + r < m)
                        def _():
                            csplat = _bcast_lane(cells, r)
                            kv = plsc.load_gather(cnt_v, [csplat - lo])
                            inv = 1.0 / (kv + 1.0)
                            a = kv * inv
                            plsc.store_scatter(cnt_v, [csplat - lo], kv + 1.0,
                                               mask=iota == 0)
                            lrr_s = _lane_scalar(lrr, r)
                            lr_s = lax.shift_right_logical(lrr_s, 3)
                            rr_s = lrr_s & 7
                            for t in range(D // L):
                                sl = pl.ds(t * L, L)
                                cbuf[lr_s, sl] = cbuf[lr_s, sl] * a + gb[r, rr_s, sl] * inv
                        return carry2

                    lax.fori_loop(0, GP, pair, 0)
            return carry

        lax.fori_loop(0, (ng + 1) // 2, gouter, 0)

    bufs = ((cb0, si0, so0), (cb1, si1, so1))

    def outer(jj, carry):
        for bsel in range(2):
            cbuf, si, so = bufs[bsel]
            ocbuf, osi, oso = bufs[1 - bsel]
            j = jj * 2 + bsel
            r0 = lo + j * CCH
            pltpu.make_async_copy(mem_h.at[pl.ds(r0, CCH)], cbuf, si).wait()

            @pl.when(j + 1 < nch)
            def _():
                @pl.when(j >= 1)
                def _():
                    pltpu.make_async_copy(
                        ocbuf, out_h.at[pl.ds(r0 - CCH, CCH)], oso).wait()
                pltpu.async_copy(mem_h.at[pl.ds(r0 + CCH, CCH)], ocbuf, osi)

            process(j, cbuf)
            pltpu.async_copy(cbuf, out_h.at[pl.ds(r0, CCH)], so)
        return carry

    lax.fori_loop(0, nch // 2, outer, 0)

    rlast = lo + (nch - 1) * CCH
    pltpu.make_async_copy(cb0, out_h.at[pl.ds(rlast - CCH, CCH)], so0).wait()
    pltpu.make_async_copy(cb1, out_h.at[pl.ds(rlast, CCH)], so1).wait()


@jax.jit
def kernel(mem, ent_counter, query, cell_idx):
    cnt_pad = jnp.pad(ent_counter, (0, NW * OWN - M))
    q3 = jnp.pad(query, ((0, 0), (0, DP - D))).reshape(B // 8, 8, DP)
    idx32 = cell_idx.astype(jnp.int32)
    mesh = plsc.VectorSubcoreMesh(core_axis_name="c", subcore_axis_name="s",
                                  num_cores=2, num_subcores=16)
    f = pl.kernel(
        _sc_body,
        out_type=jax.ShapeDtypeStruct((M, D), jnp.float32),
        mesh=mesh,
        compiler_params=pltpu.CompilerParams(needs_layout_passes=False),
        scratch_types=[
            pltpu.VMEM((ICH,), jnp.int32),
            pltpu.VMEM((B + L,), jnp.int32),
            pltpu.VMEM((B + L,), jnp.int32),
            pltpu.VMEM((OWN,), jnp.float32),
            pltpu.VMEM((CCH, D), jnp.float32),
            pltpu.VMEM((CCH, D), jnp.float32),
            pltpu.VMEM((GP, 8, DP), jnp.float32),
            pltpu.VMEM((GP, 8, DP), jnp.float32),
            pltpu.VMEM((GP,), jnp.int32),
            pltpu.VMEM((GP,), jnp.int32),
            pltpu.SemaphoreType.DMA,
            pltpu.SemaphoreType.DMA,
            pltpu.SemaphoreType.DMA,
            pltpu.SemaphoreType.DMA,
            pltpu.SemaphoreType.DMA,
            pltpu.SemaphoreType.DMA,
        ],
    )
    return f(mem, cnt_pad, q3, idx32)
